# baseline (speedup 1.0000x reference)
import functools

import jax
import jax.numpy as jnp
from jax import lax
from jax.experimental import pallas as pl
from jax.experimental.pallas import tpu as pltpu
from jax.experimental.pallas import tpu_sc as plsc

D_MODEL = 64
SCALE = 8.0
LANES = 16
BBLK = 128


@functools.lru_cache(maxsize=None)
def _build(batch, hist, vocab):
    info = plsc.get_sparse_core_info()
    nw = info.num_cores * info.num_subcores
    assert batch == nw * BBLK

    mesh = plsc.VectorSubcoreMesh(core_axis_name="c", subcore_axis_name="s")

    @functools.partial(
        pl.kernel,
        mesh=mesh,
        out_type=jax.ShapeDtypeStruct((hist, D_MODEL, batch), jnp.float32),
        scratch_types=[
            pltpu.VMEM((hist, BBLK), jnp.int32),
            pltpu.VMEM((hist, BBLK), jnp.int32),
            pltpu.VMEM((BBLK, 2 * D_MODEL), jnp.float32),
            pltpu.VMEM((BBLK, 2 * D_MODEL), jnp.float32),
            pltpu.VMEM((D_MODEL, BBLK), jnp.float32),
            pltpu.VMEM((D_MODEL, BBLK), jnp.float32),
            pltpu.SemaphoreType.DMA,
            pltpu.SemaphoreType.DMA,
        ],
        compiler_params=pltpu.CompilerParams(use_tc_tiling_on_sc=True,
                                             needs_layout_passes=False),
    )
    def k(tablep_hbm, xt_hbm, out_hbm, idx_v, pair_v, g0, g1, o0, o1,
          gsem, wsem):
        gbufs = [g0, g1]
        obufs = [o0, o1]
        wid = lax.axis_index("s") * info.num_cores + lax.axis_index("c")
        bbase = wid * BBLK
        pltpu.sync_copy(xt_hbm.at[pl.ds(0, hist), pl.ds(bbase, BBLK)], idx_v)

        def pair_body(h, carry):
            for sl in range(BBLK // LANES):
                s = pl.ds(sl * LANES, LANES)
                pair_v[h, s] = lax.shift_right_logical(idx_v[h, s], 1)
            return carry

        lax.fori_loop(0, hist, pair_body, 0)

        def start_gather(h, slot):
            return pltpu.async_copy(
                tablep_hbm.at[pair_v.at[h]], gbufs[slot], gsem)

        def wait_gather(h, slot):
            pltpu.make_async_copy(
                tablep_hbm.at[pair_v.at[h]], gbufs[slot], gsem).wait()

        def start_write(h, slot):
            return pltpu.async_copy(
                obufs[slot], out_hbm.at[h, :, pl.ds(bbase, BBLK)], wsem)

        def wait_write(h, slot):
            pltpu.make_async_copy(
                obufs[slot], out_hbm.at[h, :, pl.ds(bbase, BBLK)], wsem).wait()

        iota16 = lax.iota(jnp.int32, LANES)

        def transpose_scale(h, slot):
            gbuf = gbufs[slot]
            obuf = obufs[slot]
            for lg in range(BBLK // LANES):
                s = pl.ds(lg * LANES, LANES)
                rowi = iota16 + (lg * LANES)
                base = (idx_v[h, s] & 1) * D_MODEL

                def d_body(d, carry, rowi=rowi, base=base, s=s,
                           gbuf=gbuf, obuf=obuf):
                    vals = plsc.load_gather(gbuf, [rowi, base + d])
                    obuf[d, s] = vals * SCALE
                    return carry

                lax.fori_loop(0, D_MODEL, d_body, 0, unroll=4)

        def slot_work(h, slot, first_write, do_gather):
            if not first_write:
                wait_write(h - 2, slot)
            wait_gather(h, slot)
            transpose_scale(h, slot)
            start_write(h, slot)
            if do_gather:
                start_gather(h + 2, slot)

        start_gather(0, 0)
        start_gather(1, 1)
        slot_work(0, 0, True, True)
        slot_work(1, 1, True, True)

        def outer(o, carry):
            h = 2 * o
            slot_work(h, 0, False, True)
            slot_work(h + 1, 1, False, True)
            return carry

        lax.fori_loop(1, hist // 2 - 1, outer, 0)

        slot_work(hist - 2, 0, False, False)
        slot_work(hist - 1, 1, False, False)
        wait_write(hist - 2, 0)
        wait_write(hist - 1, 1)

    return k


def kernel(x, table):
    b, h = x.shape
    vocab = table.shape[0]
    tablep = table.reshape(vocab // 2, 2 * D_MODEL)
    out_t = _build(b, h, vocab)(tablep, x.T.astype(jnp.int32))
    return out_t.transpose(2, 0, 1)
